# baseline (speedup 1.0000x reference)
import jax
import jax.numpy as jnp
from jax import lax
from jax.experimental import pallas as pl
from jax.experimental.pallas import tpu as pltpu
from jax.experimental.pallas import tpu_sc as plsc

_SEQ = 8192
_DIM = 1024
_NC = 2
_NS = 16
_NW = _NC * _NS
_ROWS_PER_W = _SEQ // _NW
_NBUF = 3
_CH = 32
_NCH = _ROWS_PER_W // _CH


def _copy_body(table_hbm, out_hbm, *scratch):
    bufs = scratch[:_NBUF]
    isems = scratch[_NBUF:2 * _NBUF]
    osems = scratch[2 * _NBUF:]
    wid = lax.axis_index("s") * _NC + lax.axis_index("c")
    base = wid * _ROWS_PER_W

    def in_copy(c):
        b = c % _NBUF
        return pltpu.make_async_copy(
            table_hbm.at[pl.ds(base + c * _CH, _CH)], bufs[b], isems[b])

    def out_copy(c):
        b = c % _NBUF
        return pltpu.make_async_copy(
            bufs[b], out_hbm.at[pl.ds(base + c * _CH, _CH)], osems[b])

    for c in range(_NBUF):
        in_copy(c).start()
    for c in range(_NCH):
        in_copy(c).wait()
        out_copy(c).start()
        if c + _NBUF < _NCH:
            out_copy(c).wait()
            in_copy(c + _NBUF).start()
    for c in range(_NCH - _NBUF, _NCH):
        out_copy(c).wait()


def kernel(x, emb_weight):
    del x
    mesh = plsc.VectorSubcoreMesh(core_axis_name="c", subcore_axis_name="s")
    run = pl.kernel(
        _copy_body,
        mesh=mesh,
        out_type=jax.ShapeDtypeStruct((_SEQ, _DIM), jnp.float32),
        scratch_types=(
            [pltpu.VMEM((_CH, _DIM), jnp.float32) for _ in range(_NBUF)]
            + [pltpu.SemaphoreType.DMA for _ in range(2 * _NBUF)]
        ),
    )
    return run(emb_weight)


# speedup vs baseline: 1.0007x; 1.0007x over previous
import jax
import jax.numpy as jnp
from jax import lax
from jax.experimental import pallas as pl
from jax.experimental.pallas import tpu as pltpu
from jax.experimental.pallas import tpu_sc as plsc

_SEQ = 8192
_DIM = 1024
_NC = 2
_NS = 16
_NW = _NC * _NS
_ROWS_PER_W = _SEQ // _NW
_NBUF = 2
_CH = 56
_CHUNKS = [(i * _CH, _CH) for i in range(4)] + [(4 * _CH, _ROWS_PER_W - 4 * _CH)]
_NCH = len(_CHUNKS)


def _copy_body(table_hbm, out_hbm, *scratch):
    bufs = scratch[:_NBUF]
    isems = scratch[_NBUF:2 * _NBUF]
    osems = scratch[2 * _NBUF:]
    wid = lax.axis_index("s") * _NC + lax.axis_index("c")
    base = wid * _ROWS_PER_W

    def in_copy(c):
        b = c % _NBUF
        off, ln = _CHUNKS[c]
        return pltpu.make_async_copy(
            table_hbm.at[pl.ds(base + off, ln)],
            bufs[b].at[pl.ds(0, ln)], isems[b])

    def out_copy(c):
        b = c % _NBUF
        off, ln = _CHUNKS[c]
        return pltpu.make_async_copy(
            bufs[b].at[pl.ds(0, ln)],
            out_hbm.at[pl.ds(base + off, ln)], osems[b])

    for c in range(min(_NBUF, _NCH)):
        in_copy(c).start()
    for c in range(_NCH):
        in_copy(c).wait()
        out_copy(c).start()
        if c + _NBUF < _NCH:
            out_copy(c).wait()
            in_copy(c + _NBUF).start()
    for c in range(max(0, _NCH - _NBUF), _NCH):
        out_copy(c).wait()


def kernel(x, emb_weight):
    del x
    mesh = plsc.VectorSubcoreMesh(core_axis_name="c", subcore_axis_name="s")
    run = pl.kernel(
        _copy_body,
        mesh=mesh,
        out_type=jax.ShapeDtypeStruct((_SEQ, _DIM), jnp.float32),
        scratch_types=(
            [pltpu.VMEM((_CH, _DIM), jnp.float32) for _ in range(_NBUF)]
            + [pltpu.SemaphoreType.DMA for _ in range(2 * _NBUF)]
        ),
    )
    return run(emb_weight)
